# Initial kernel scaffold; baseline (speedup 1.0000x reference)
import jax
import jax.numpy as jnp
from jax.experimental import pallas as pl


def kernel(x, weight):
    raise NotImplementedError("write your pallas kernel here")


# speedup vs baseline: 2.8759x; 2.8759x over previous
import functools

import jax
import jax.numpy as jnp
from jax import lax
from jax.experimental import pallas as pl
from jax.experimental.pallas import tpu as pltpu
from jax.experimental.pallas import tpu_sc as plsc

NUM_EMB = 100000
D = 128
SCALE_F = 10.0
B = 4096 * 50
NC, NS, L = 2, 16, 16
NW = NC * NS
B_PER_W = B // NW
CH = 128
NCHUNK = B_PER_W // CH


def _sc_gather_scale(table, idx):
    mesh = plsc.VectorSubcoreMesh(core_axis_name="c", subcore_axis_name="s")

    @functools.partial(
        pl.kernel,
        mesh=mesh,
        out_type=jax.ShapeDtypeStruct((B, D), jnp.float32),
        scratch_types=[
            pltpu.VMEM((B_PER_W,), jnp.int32),
            pltpu.VMEM((CH, D), jnp.float32),
            pltpu.VMEM((CH, D), jnp.float32),
            pltpu.SemaphoreType.DMA,
            pltpu.SemaphoreType.DMA,
        ],
    )
    def k(table_hbm, idx_hbm, out_hbm, idx_v, rows0, rows1, sem0, sem1):
        wid = lax.axis_index("s") * NC + lax.axis_index("c")
        base = wid * B_PER_W

        pltpu.sync_copy(idx_hbm.at[pl.ds(base, B_PER_W)], idx_v)

        rows = (rows0, rows1)
        sems = (sem0, sem1)

        def gather(c, buf, sem):
            pltpu.async_copy(
                table_hbm.at[idx_v.at[pl.ds(c * CH, CH)]], buf, sem)

        gather(0, rows0, sem0)
        gather(1, rows1, sem1)

        def step(c0, _):
            for b in range(2):
                c = c0 + b
                buf, sem = rows[b], sems[b]
                pltpu.make_async_copy(
                    table_hbm.at[idx_v.at[pl.ds(0, CH)]], buf, sem).wait()

                def scale_row(r, _):
                    for j in range(D // L):
                        s = pl.ds(j * L, L)
                        buf[r, s] = buf[r, s] * SCALE_F
                    return ()
                lax.fori_loop(0, CH, scale_row, (), unroll=False)

                pltpu.sync_copy(buf, out_hbm.at[pl.ds(base + c * CH, CH)])

                @pl.when(c + 2 < NCHUNK)
                def _():
                    gather(c + 2, buf, sem)
            return ()

        lax.fori_loop(0, NCHUNK // 2, lambda i, a: step(i * 2, a), ())

    return k(table, idx)


def kernel(x, weight):
    idx = x.reshape(-1).astype(jnp.int32)
    out = _sc_gather_scale(weight, idx)
    return out.reshape(x.shape + (D,))
